# baseline (speedup 1.0000x reference)
import jax
import jax.numpy as jnp
from jax import lax
from jax.experimental import pallas as pl

N_BINS = 64
EPS = 1e-06
LOW = 0.0
HIGH = 1.0


def _tokenize_block(x_ref, e_ref, c_ref, tokens_ref, mask_ref):
    x = jnp.clip(x_ref[...], LOW + EPS, HIGH - EPS)
    g = jax.lax.dot(x, e_ref[...], precision=lax.Precision.HIGHEST,
                    preferred_element_type=jnp.float32)
    tokens_ref[...] = (jnp.floor(g) == c_ref[0:1, :]).astype(jnp.float32)
    mask_ref[...] = jnp.ones(mask_ref.shape, jnp.float32)


@jax.jit
def kernel(observations):
    B, T, D = observations.shape
    R = B * T
    W = D * N_BINS
    bR = 512
    x2 = observations.reshape(R, D)
    j = lax.broadcasted_iota(jnp.int32, (D, W), 1)
    d = lax.broadcasted_iota(jnp.int32, (D, W), 0)
    e = jnp.where(j // N_BINS == d, jnp.float32(N_BINS), 0.0)
    c = (lax.broadcasted_iota(jnp.int32, (8, W), 1) % N_BINS).astype(jnp.float32)
    grid = (R // bR,)
    tokens, mask = pl.pallas_call(
        _tokenize_block,
        grid=grid,
        in_specs=[
            pl.BlockSpec((bR, D), lambda i: (i, 0)),
            pl.BlockSpec((D, W), lambda i: (0, 0)),
            pl.BlockSpec((8, W), lambda i: (0, 0)),
        ],
        out_specs=[
            pl.BlockSpec((bR, W), lambda i: (i, 0)),
            pl.BlockSpec((bR, D), lambda i: (i, 0)),
        ],
        out_shape=[
            jax.ShapeDtypeStruct((R, W), jnp.float32),
            jax.ShapeDtypeStruct((R, D), jnp.float32),
        ],
    )(x2, e, c)
    return (tokens.reshape(B, T, D, N_BINS), mask.reshape(B, T, D))


# speedup vs baseline: 1.6693x; 1.6693x over previous
import jax
import jax.numpy as jnp
from jax import lax
from jax.experimental import pallas as pl

N_BINS = 64
EPS = 1e-06
LOW = 0.0
HIGH = 1.0


def _tokenize_block(obs_ref, tokens_ref, mask_ref):
    x = obs_ref[...]
    x = jnp.clip(x, LOW + EPS, HIGH - EPS)
    bins = jnp.floor(x * N_BINS).astype(jnp.int32)
    iota = lax.broadcasted_iota(jnp.int32, tokens_ref.shape, 3)
    tokens_ref[...] = (bins[..., None] == iota).astype(jnp.float32)
    mask_ref[...] = jnp.ones(mask_ref.shape, jnp.float32)


@jax.jit
def kernel(observations):
    B, T, D = observations.shape
    bB = 64
    grid = (B // bB,)
    tokens, mask = pl.pallas_call(
        _tokenize_block,
        grid=grid,
        in_specs=[pl.BlockSpec((bB, T, D), lambda i: (i, 0, 0))],
        out_specs=[
            pl.BlockSpec((bB, T, D, N_BINS), lambda i: (i, 0, 0, 0)),
            pl.BlockSpec((bB, T, D), lambda i: (i, 0, 0)),
        ],
        out_shape=[
            jax.ShapeDtypeStruct((B, T, D, N_BINS), jnp.float32),
            jax.ShapeDtypeStruct((B, T, D), jnp.float32),
        ],
    )(observations)
    return (tokens, mask)


# speedup vs baseline: 1.8300x; 1.0963x over previous
import jax
import jax.numpy as jnp
from jax import lax
from jax.experimental import pallas as pl

N_BINS = 64
EPS = 1e-06
LOW = 0.0
HIGH = 1.0


def _tokenize_block(x_ref, e_ref, c_ref, tokens_ref, mask_ref):
    bB, T, D = x_ref.shape
    W = D * N_BINS
    x = jnp.clip(x_ref[...], LOW + EPS, HIGH - EPS).reshape(bB * T, D)
    g = jax.lax.dot(x, e_ref[...], precision=lax.Precision.HIGHEST,
                    preferred_element_type=jnp.float32)
    tokens_ref[...] = (jnp.floor(g) == c_ref[0:1, :]).astype(jnp.float32).reshape(
        bB, T, W)
    mask_ref[...] = jnp.ones(mask_ref.shape, jnp.float32)


@jax.jit
def kernel(observations):
    B, T, D = observations.shape
    W = D * N_BINS
    bB = 32
    j = lax.broadcasted_iota(jnp.int32, (D, W), 1)
    d = lax.broadcasted_iota(jnp.int32, (D, W), 0)
    e = jnp.where(j // N_BINS == d, jnp.float32(N_BINS), 0.0)
    c = (lax.broadcasted_iota(jnp.int32, (8, W), 1) % N_BINS).astype(jnp.float32)
    grid = (B // bB,)
    tokens, mask = pl.pallas_call(
        _tokenize_block,
        grid=grid,
        in_specs=[
            pl.BlockSpec((bB, T, D), lambda i: (i, 0, 0)),
            pl.BlockSpec((D, W), lambda i: (0, 0)),
            pl.BlockSpec((8, W), lambda i: (0, 0)),
        ],
        out_specs=[
            pl.BlockSpec((bB, T, W), lambda i: (i, 0, 0)),
            pl.BlockSpec((bB, T, D), lambda i: (i, 0, 0)),
        ],
        out_shape=[
            jax.ShapeDtypeStruct((B, T, W), jnp.float32),
            jax.ShapeDtypeStruct((B, T, D), jnp.float32),
        ],
    )(observations, e, c)
    return (tokens.reshape(B, T, D, N_BINS), mask)
